# baseline (speedup 1.0000x reference)
import functools

import jax
import jax.numpy as jnp
from jax import lax
from jax.experimental import pallas as pl
from jax.experimental.pallas import tpu as pltpu
from jax.experimental.pallas import tpu_sc as plsc

N = 10000
D = 256
E = 160000
NC = 2
NS = 16
DH = D // NC
EPT = E // NS
C = 125
NCH = EPT // C
NPH = 2
NCH2 = NCH // NPH
RPT = 624
REM = N - NS * RPT
BM = 2000

def _deg_body(dst_hbm, z1d_hbm, deg_hbm, dst_v, ones_v, deg_sh):
    cid = lax.axis_index("c")
    sid = lax.axis_index("s")
    one = jnp.full((16,), 1.0, jnp.float32)
    for i in range(8):
        ones_v[pl.ds(i * 16, 16)] = one

    @pl.when(sid == 0)
    def _():
        pltpu.sync_copy(z1d_hbm, deg_sh)

    pltpu.sync_copy(dst_hbm.at[cid * NS + sid], dst_v)
    plsc.subcore_barrier()

    def body(j, carry):
        pltpu.sync_copy(ones_v.at[pl.ds(0, C)],
                        deg_sh.at[dst_v.at[j]], add=True)
        return carry

    lax.fori_loop(0, NCH2, body, 0)
    plsc.subcore_barrier()

    @pl.when(sid == 0)
    def _():
        pltpu.sync_copy(deg_sh, deg_hbm.at[cid])


def _msg_body(table_hbm, src_hbm, dst_hbm, zblk_hbm, out_hbm,
              idx_v, dst_v, rows_v, agg_sh, sem0, sem1, ssem0, ssem1):
    cid = lax.axis_index("c")
    sid = lax.axis_index("s")
    tbl = table_hbm.at[cid]
    pltpu.sync_copy(zblk_hbm, agg_sh.at[pl.ds(sid * RPT, RPT)])

    @pl.when(sid == 0)
    def _():
        pltpu.sync_copy(zblk_hbm.at[pl.ds(0, REM)],
                        agg_sh.at[pl.ds(NS * RPT, REM)])

    def body(jj, carry):
        j0 = 2 * jj
        j1 = j0 + 1
        pltpu.make_async_copy(
            tbl.at[idx_v.at[j0]], rows_v.at[0], sem0).wait()
        pltpu.async_copy(rows_v.at[0], agg_sh.at[dst_v.at[j0]], ssem0,
                         add=True)
        pltpu.make_async_copy(
            tbl.at[idx_v.at[j1]], rows_v.at[1], sem1).wait()
        pltpu.async_copy(rows_v.at[1], agg_sh.at[dst_v.at[j1]], ssem1,
                         add=True)

        @pl.when(j0 + 2 < NCH2)
        def _():
            pltpu.make_async_copy(
                rows_v.at[0], agg_sh.at[dst_v.at[j0]], ssem0).wait()
            pltpu.async_copy(tbl.at[idx_v.at[j0 + 2]], rows_v.at[0], sem0)

        @pl.when(j1 + 2 < NCH2)
        def _():
            pltpu.make_async_copy(
                rows_v.at[1], agg_sh.at[dst_v.at[j1]], ssem1).wait()
            pltpu.async_copy(tbl.at[idx_v.at[j1 + 2]], rows_v.at[1], sem1)

        return carry

    for phase in range(NPH):
        pltpu.sync_copy(src_hbm.at[sid * NPH + phase], idx_v)
        pltpu.sync_copy(dst_hbm.at[sid * NPH + phase], dst_v)
        if phase == 0:
            plsc.subcore_barrier()
        pltpu.async_copy(tbl.at[idx_v.at[0]], rows_v.at[0], sem0)
        pltpu.async_copy(tbl.at[idx_v.at[1]], rows_v.at[1], sem1)
        lax.fori_loop(0, NCH2 // 2, body, 0)
        pltpu.make_async_copy(
            rows_v.at[0], agg_sh.at[dst_v.at[NCH2 - 2]], ssem0).wait()
        pltpu.make_async_copy(
            rows_v.at[1], agg_sh.at[dst_v.at[NCH2 - 1]], ssem1).wait()
    plsc.subcore_barrier()
    pltpu.sync_copy(agg_sh.at[pl.ds(sid * RPT, RPT)],
                    out_hbm.at[pl.ds(cid * N + sid * RPT, RPT)])

    @pl.when(sid == 0)
    def _():
        pltpu.sync_copy(agg_sh.at[pl.ds(NS * RPT, REM)],
                        out_hbm.at[pl.ds(cid * N + NS * RPT, REM)])


@functools.cache
def _sc_kernels():
    mesh = plsc.VectorSubcoreMesh(
        core_axis_name="c", subcore_axis_name="s",
        num_cores=NC, num_subcores=NS)
    deg_kernel = pl.kernel(
        _deg_body,
        out_type=jax.ShapeDtypeStruct((NC, N), jnp.float32),
        mesh=mesh,
        scratch_types=[
            pltpu.VMEM((NCH2, C), jnp.int32),
            pltpu.VMEM((128,), jnp.float32),
            pltpu.VMEM_SHARED((N,), jnp.float32),
        ],
    )
    msg_kernel = pl.kernel(
        _msg_body,
        out_type=jax.ShapeDtypeStruct((NC * N, DH), jnp.float32),
        mesh=mesh,
        scratch_types=[
            pltpu.VMEM((NCH2, C), jnp.int32),
            pltpu.VMEM((NCH2, C), jnp.int32),
            pltpu.VMEM((2, C, DH), jnp.float32),
            pltpu.VMEM_SHARED((N, DH), jnp.float32),
            pltpu.SemaphoreType.DMA,
            pltpu.SemaphoreType.DMA,
            pltpu.SemaphoreType.DMA,
            pltpu.SemaphoreType.DMA,
        ],
    )
    return deg_kernel, msg_kernel


def _mm1_body(x_ref, w_ref, deg_ref, out_ref, dinv_ref):
    dinv = lax.rsqrt(deg_ref[0] + deg_ref[1] + 1.0)
    hw = jnp.dot(x_ref[...], w_ref[...],
                 preferred_element_type=jnp.float32)
    out_ref[...] = (hw * dinv)[None]
    dinv_ref[...] = dinv


_mm1 = pl.pallas_call(
    _mm1_body,
    grid=(N // BM, NC),
    in_specs=[
        pl.BlockSpec((BM, D), lambda i, c: (i, 0)),
        pl.BlockSpec((D, DH), lambda i, c: (0, c)),
        pl.BlockSpec((NC, BM, 1), lambda i, c: (0, i, 0)),
    ],
    out_specs=[
        pl.BlockSpec((1, BM, DH), lambda i, c: (c, i, 0)),
        pl.BlockSpec((BM, 1), lambda i, c: (i, 0)),
    ],
    out_shape=[
        jax.ShapeDtypeStruct((NC, N, DH), jnp.float32),
        jax.ShapeDtypeStruct((N, 1), jnp.float32),
    ],
)


def _selu(v):
    alpha = 1.6732632423543772
    scale = 1.0507009873554805
    return scale * jnp.where(
        v > 0, v, alpha * (jnp.exp(jnp.minimum(v, 0.0)) - 1.0))


def _mm2_body(s_ref, hwp_ref, dinv_ref, b_ref, w_ref, out_ref):
    dinv = dinv_ref[...]
    t = (s_ref[...] + hwp_ref[...]) * dinv[None]
    h0 = _selu(t[0] + b_ref[0, :DH][None, :])
    h1 = _selu(t[1] + b_ref[0, DH:][None, :])
    acc = jnp.dot(h0, w_ref[0], preferred_element_type=jnp.float32)
    acc = acc + jnp.dot(h1, w_ref[1], preferred_element_type=jnp.float32)
    out_ref[...] = (acc * dinv)[None]


_mm2 = pl.pallas_call(
    _mm2_body,
    grid=(N // BM, NC),
    in_specs=[
        pl.BlockSpec((NC, BM, DH), lambda i, c: (0, i, 0)),
        pl.BlockSpec((NC, BM, DH), lambda i, c: (0, i, 0)),
        pl.BlockSpec((BM, 1), lambda i, c: (i, 0)),
        pl.BlockSpec((1, D), lambda i, c: (0, 0)),
        pl.BlockSpec((NC, DH, DH), lambda i, c: (0, 0, c)),
    ],
    out_specs=pl.BlockSpec((1, BM, DH), lambda i, c: (c, i, 0)),
    out_shape=jax.ShapeDtypeStruct((NC, N, DH), jnp.float32),
)


def _final_body(s_ref, hwp_ref, dinv_ref, b_ref, out_ref):
    dinv = dinv_ref[...]
    t = (s_ref[...] + hwp_ref[...]) * dinv[None]
    out_ref[:, :DH] = t[0] + b_ref[0, :DH][None, :]
    out_ref[:, DH:] = t[1] + b_ref[0, DH:][None, :]


_final = pl.pallas_call(
    _final_body,
    grid=(N // BM,),
    in_specs=[
        pl.BlockSpec((NC, BM, DH), lambda i: (0, i, 0)),
        pl.BlockSpec((NC, BM, DH), lambda i: (0, i, 0)),
        pl.BlockSpec((BM, 1), lambda i: (i, 0)),
        pl.BlockSpec((1, D), lambda i: (0, 0)),
    ],
    out_specs=pl.BlockSpec((BM, D), lambda i: (i, 0)),
    out_shape=jax.ShapeDtypeStruct((N, D), jnp.float32),
)


@jax.jit
def kernel(x, adj_t, W1, b1, W2, b2):
    _deg_kernel, _msg_kernel = _sc_kernels()
    src = adj_t[0]
    dst = adj_t[1]
    dst_r = dst.reshape(NS * NPH, NCH2, C)
    src_r = src.reshape(NS * NPH, NCH2, C)
    z1d = jnp.zeros((N,), jnp.float32)
    zblk = jnp.zeros((RPT, DH), jnp.float32)

    deg = _deg_kernel(dst_r, z1d)
    deg3 = deg.reshape(NC, N, 1)

    hw1p, dinv2d = _mm1(x, W1, deg3)
    S1 = _msg_kernel(hw1p, src_r, dst_r, zblk)
    hw2p = _mm2(S1.reshape(NC, N, DH), hw1p, dinv2d,
                b1.reshape(1, D), W2.reshape(NC, DH, D))
    S2 = _msg_kernel(hw2p, src_r, dst_r, zblk)
    out = _final(S2.reshape(NC, N, DH), hw2p, dinv2d, b2.reshape(1, D))
    return out


# speedup vs baseline: 1.2268x; 1.2268x over previous
import functools

import jax
import jax.numpy as jnp
from jax import lax
from jax.experimental import pallas as pl
from jax.experimental.pallas import tpu as pltpu
from jax.experimental.pallas import tpu_sc as plsc

N = 10000
D = 256
E = 160000
NC = 2
NS = 16
DH = D // NC
EPT = E // NS
C = 125
NCH = EPT // C
NPH = 2
NCH2 = NCH // NPH
RPT = 624
REM = N - NS * RPT
BM = 2000

def _deg_body(dst_hbm, z1d_hbm, deg_hbm, dst_v, ones_v, deg_sh):
    cid = lax.axis_index("c")
    sid = lax.axis_index("s")
    one = jnp.full((16,), 1.0, jnp.float32)
    for i in range(8):
        ones_v[pl.ds(i * 16, 16)] = one

    @pl.when(sid == 0)
    def _():
        pltpu.sync_copy(z1d_hbm, deg_sh)

    pltpu.sync_copy(dst_hbm.at[cid * NS + sid], dst_v)
    plsc.subcore_barrier()

    def body(j, carry):
        pltpu.sync_copy(ones_v.at[pl.ds(0, C)],
                        deg_sh.at[dst_v.at[j]], add=True)
        return carry

    lax.fori_loop(0, NCH2, body, 0)
    plsc.subcore_barrier()

    @pl.when(sid == 0)
    def _():
        pltpu.sync_copy(deg_sh, deg_hbm.at[cid])


def _msg_body(table_hbm, srcx_hbm, dst_hbm, zblk_hbm, out_hbm,
              idx_v, dst_v, rows_v, agg_sh, sem0, sem1):
    cid = lax.axis_index("c")
    sid = lax.axis_index("s")
    wid = cid * NS + sid
    pltpu.sync_copy(zblk_hbm, agg_sh.at[pl.ds(sid * RPT, RPT)])

    @pl.when(sid == 0)
    def _():
        pltpu.sync_copy(zblk_hbm.at[pl.ds(0, REM)],
                        agg_sh.at[pl.ds(NS * RPT, REM)])

    def body(jj, carry):
        j0 = 2 * jj
        j1 = j0 + 1
        pltpu.make_async_copy(
            table_hbm.at[idx_v.at[j0]], rows_v.at[0], sem0).wait()
        pltpu.sync_copy(rows_v.at[0], agg_sh.at[dst_v.at[j0]], add=True)

        @pl.when(j0 + 2 < NCH2)
        def _():
            pltpu.async_copy(
                table_hbm.at[idx_v.at[j0 + 2]], rows_v.at[0], sem0)

        pltpu.make_async_copy(
            table_hbm.at[idx_v.at[j1]], rows_v.at[1], sem1).wait()
        pltpu.sync_copy(rows_v.at[1], agg_sh.at[dst_v.at[j1]], add=True)

        @pl.when(j1 + 2 < NCH2)
        def _():
            pltpu.async_copy(
                table_hbm.at[idx_v.at[j1 + 2]], rows_v.at[1], sem1)

        return carry

    for phase in range(NPH):
        pltpu.sync_copy(srcx_hbm.at[wid * NPH + phase], idx_v)
        pltpu.sync_copy(dst_hbm.at[sid * NPH + phase], dst_v)
        if phase == 0:
            plsc.subcore_barrier()
        pltpu.async_copy(table_hbm.at[idx_v.at[0]], rows_v.at[0], sem0)
        pltpu.async_copy(table_hbm.at[idx_v.at[1]], rows_v.at[1], sem1)
        lax.fori_loop(0, NCH2 // 2, body, 0)
    plsc.subcore_barrier()
    pltpu.sync_copy(agg_sh.at[pl.ds(sid * RPT, RPT)],
                    out_hbm.at[pl.ds(cid * N + sid * RPT, RPT)])

    @pl.when(sid == 0)
    def _():
        pltpu.sync_copy(agg_sh.at[pl.ds(NS * RPT, REM)],
                        out_hbm.at[pl.ds(cid * N + NS * RPT, REM)])


@functools.cache
def _sc_kernels():
    mesh = plsc.VectorSubcoreMesh(
        core_axis_name="c", subcore_axis_name="s",
        num_cores=NC, num_subcores=NS)
    deg_kernel = pl.kernel(
        _deg_body,
        out_type=jax.ShapeDtypeStruct((NC, N), jnp.float32),
        mesh=mesh,
        scratch_types=[
            pltpu.VMEM((NCH2, C), jnp.int32),
            pltpu.VMEM((128,), jnp.float32),
            pltpu.VMEM_SHARED((N,), jnp.float32),
        ],
    )
    msg_kernel = pl.kernel(
        _msg_body,
        out_type=jax.ShapeDtypeStruct((NC * N, DH), jnp.float32),
        mesh=mesh,
        scratch_types=[
            pltpu.VMEM((NCH2, C), jnp.int32),
            pltpu.VMEM((NCH2, C), jnp.int32),
            pltpu.VMEM((2, C, DH), jnp.float32),
            pltpu.VMEM_SHARED((N, DH), jnp.float32),
            pltpu.SemaphoreType.DMA,
            pltpu.SemaphoreType.DMA,
        ],
    )
    return deg_kernel, msg_kernel


def _dinv_of(deg_ref):
    return lax.rsqrt(deg_ref[0] + deg_ref[1] + 1.0)


def _mm1_body(x_ref, w_ref, deg_ref, out_ref, dinv_ref):
    dinv = _dinv_of(deg_ref)
    hw = jnp.dot(x_ref[...], w_ref[...],
                 preferred_element_type=jnp.float32)
    out_ref[...] = (hw * dinv)[None]
    dinv_ref[...] = dinv


_mm1 = pl.pallas_call(
    _mm1_body,
    grid=(N // BM, NC),
    in_specs=[
        pl.BlockSpec((BM, D), lambda i, c: (i, 0)),
        pl.BlockSpec((D, DH), lambda i, c: (0, c)),
        pl.BlockSpec((NC, BM, 1), lambda i, c: (0, i, 0)),
    ],
    out_specs=[
        pl.BlockSpec((1, BM, DH), lambda i, c: (c, i, 0)),
        pl.BlockSpec((BM, 1), lambda i, c: (i, 0)),
    ],
    out_shape=[
        jax.ShapeDtypeStruct((NC, N, DH), jnp.float32),
        jax.ShapeDtypeStruct((N, 1), jnp.float32),
    ],
)


def _selu(v):
    alpha = 1.6732632423543772
    scale = 1.0507009873554805
    return scale * jnp.where(
        v > 0, v, alpha * (jnp.exp(jnp.minimum(v, 0.0)) - 1.0))


def _mm2_body(s_ref, hwp_ref, dinv_ref, b_ref, w_ref, out_ref):
    dinv = dinv_ref[...]
    t = (s_ref[...] + hwp_ref[...]) * dinv[None]
    h0 = _selu(t[0] + b_ref[0, :DH][None, :])
    h1 = _selu(t[1] + b_ref[0, DH:][None, :])
    acc = jnp.dot(h0, w_ref[0], preferred_element_type=jnp.float32)
    acc = acc + jnp.dot(h1, w_ref[1], preferred_element_type=jnp.float32)
    out_ref[...] = (acc * dinv)[None]


_mm2 = pl.pallas_call(
    _mm2_body,
    grid=(N // BM, NC),
    in_specs=[
        pl.BlockSpec((NC, BM, DH), lambda i, c: (0, i, 0)),
        pl.BlockSpec((NC, BM, DH), lambda i, c: (0, i, 0)),
        pl.BlockSpec((BM, 1), lambda i, c: (i, 0)),
        pl.BlockSpec((1, D), lambda i, c: (0, 0)),
        pl.BlockSpec((NC, DH, DH), lambda i, c: (0, 0, c)),
    ],
    out_specs=pl.BlockSpec((1, BM, DH), lambda i, c: (c, i, 0)),
    out_shape=jax.ShapeDtypeStruct((NC, N, DH), jnp.float32),
)


def _final_body(s_ref, hwp_ref, dinv_ref, b_ref, out_ref):
    dinv = dinv_ref[...]
    t = (s_ref[...] + hwp_ref[...]) * dinv[None]
    out_ref[:, :DH] = t[0] + b_ref[0, :DH][None, :]
    out_ref[:, DH:] = t[1] + b_ref[0, DH:][None, :]


_final = pl.pallas_call(
    _final_body,
    grid=(N // BM,),
    in_specs=[
        pl.BlockSpec((NC, BM, DH), lambda i: (0, i, 0)),
        pl.BlockSpec((NC, BM, DH), lambda i: (0, i, 0)),
        pl.BlockSpec((BM, 1), lambda i: (i, 0)),
        pl.BlockSpec((1, D), lambda i: (0, 0)),
    ],
    out_specs=pl.BlockSpec((BM, D), lambda i: (i, 0)),
    out_shape=jax.ShapeDtypeStruct((N, D), jnp.float32),
)


@jax.jit
def kernel(x, adj_t, W1, b1, W2, b2):
    _deg_kernel, _msg_kernel = _sc_kernels()
    src = adj_t[0]
    dst = adj_t[1]
    dst_r = dst.reshape(NS * NPH, NCH2, C)
    sr = src.reshape(NS, NPH, NCH2, C)
    srcx = jnp.concatenate([sr, sr + N], axis=0)
    srcx = srcx.reshape(NC * NS * NPH, NCH2, C)
    z1d = jnp.zeros((N,), jnp.float32)
    zblk = jnp.zeros((RPT, DH), jnp.float32)

    deg = _deg_kernel(dst_r, z1d)
    deg3 = deg.reshape(NC, N, 1)

    hw1p, dinv2d = _mm1(x, W1, deg3)
    S1 = _msg_kernel(hw1p.reshape(NC * N, DH), srcx, dst_r, zblk)
    hw2p = _mm2(S1.reshape(NC, N, DH), hw1p, dinv2d,
                b1.reshape(1, D), W2.reshape(NC, DH, D))
    S2 = _msg_kernel(hw2p.reshape(NC * N, DH), srcx, dst_r, zblk)
    out = _final(S2.reshape(NC, N, DH), hw2p, dinv2d, b2.reshape(1, D))
    return out
